# Initial kernel scaffold; baseline (speedup 1.0000x reference)
import jax
import jax.numpy as jnp
from jax.experimental import pallas as pl


def kernel(x, edge_index, W1, a_src1, a_dst1, b1, W2, a_src2, a_dst2, b2, Wfc, bfc):
    raise NotImplementedError("write your pallas kernel here")


# speedup vs baseline: 27.2226x; 27.2226x over previous
import functools

import jax
import jax.numpy as jnp
from jax import lax
from jax.experimental import pallas as pl
from jax.experimental.pallas import tpu as pltpu
from jax.experimental.pallas import tpu_sc as plsc

N = 10000
E = 320000
IN = 128
HID = 16
HEADS = 8
OUT = 64
HC1 = HEADS * HID
HP = 16

NC = 2
NS = 16
NW = NC * NS
EPW = E // NW
CH = 40
NCHUNK = EPW // CH
CH2 = 80
NCHUNK2 = EPW // CH2
ROWS_PER_STAGER = 1000

_f32 = jnp.float32


BR = 400


def _dense1_body(x_ref, w_ref, asrc_ref, adst_ref, h_ref, as_ref, ad_ref):
    h = jnp.dot(x_ref[...], w_ref[...], preferred_element_type=_f32)
    h_ref[...] = h
    as_ref[...] = jnp.dot(h, asrc_ref[...], preferred_element_type=_f32,
                          precision=lax.Precision.HIGHEST)
    ad_ref[...] = jnp.dot(h, adst_ref[...], preferred_element_type=_f32,
                          precision=lax.Precision.HIGHEST)


def _dense1(x, W1, A1s, A1d):
    return pl.pallas_call(
        _dense1_body,
        grid=(N // BR,),
        in_specs=[
            pl.BlockSpec((BR, IN), lambda i: (i, 0)),
            pl.BlockSpec((IN, HC1), lambda i: (0, 0)),
            pl.BlockSpec((HC1, HP), lambda i: (0, 0)),
            pl.BlockSpec((HC1, HP), lambda i: (0, 0)),
        ],
        out_specs=[
            pl.BlockSpec((BR, HC1), lambda i: (i, 0)),
            pl.BlockSpec((BR, HP), lambda i: (i, 0)),
            pl.BlockSpec((BR, HP), lambda i: (i, 0)),
        ],
        out_shape=[
            jax.ShapeDtypeStruct((N, HC1), _f32),
            jax.ShapeDtypeStruct((N, HP), _f32),
            jax.ShapeDtypeStruct((N, HP), _f32),
        ],
    )(x, W1, A1s, A1d)


def _dense2_body(p0_ref, p1_ref, b1_ref, w2_ref, avs_ref, avd_ref,
                 h2_ref, as_ref, ad_ref):
    x2 = jnp.maximum(p0_ref[...] + p1_ref[...] + b1_ref[...], 0.0)
    h2 = jnp.dot(x2, w2_ref[...], preferred_element_type=_f32)
    h2_ref[...] = h2
    as_ref[...] = jnp.dot(h2, avs_ref[...], preferred_element_type=_f32,
                          precision=lax.Precision.HIGHEST)
    ad_ref[...] = jnp.dot(h2, avd_ref[...], preferred_element_type=_f32,
                          precision=lax.Precision.HIGHEST)


def _dense2(p0, p1, b1, W2, avs, avd):
    return pl.pallas_call(
        _dense2_body,
        grid=(N // BR,),
        in_specs=[
            pl.BlockSpec((BR, HC1), lambda i: (i, 0)),
            pl.BlockSpec((BR, HC1), lambda i: (i, 0)),
            pl.BlockSpec((1, HC1), lambda i: (0, 0)),
            pl.BlockSpec((HC1, OUT), lambda i: (0, 0)),
            pl.BlockSpec((OUT, 1), lambda i: (0, 0)),
            pl.BlockSpec((OUT, 1), lambda i: (0, 0)),
        ],
        out_specs=[
            pl.BlockSpec((BR, OUT), lambda i: (i, 0)),
            pl.BlockSpec((BR, 1), lambda i: (i, 0)),
            pl.BlockSpec((BR, 1), lambda i: (i, 0)),
        ],
        out_shape=[
            jax.ShapeDtypeStruct((N, OUT), _f32),
            jax.ShapeDtypeStruct((N, 1), _f32),
            jax.ShapeDtypeStruct((N, 1), _f32),
        ],
    )(p0, p1, b1, W2, avs, avd)


def _final_body(p0_ref, p1_ref, b2_ref, wfc_ref, bfc_ref, out_ref):
    y = p0_ref[...] + p1_ref[...] + b2_ref[...]
    out_ref[...] = jnp.dot(y, wfc_ref[...], preferred_element_type=_f32) + bfc_ref[...]


def _final(p0, p1, b2, Wfc, bfc):
    return pl.pallas_call(
        _final_body,
        grid=(N // BR,),
        in_specs=[
            pl.BlockSpec((BR, OUT), lambda i: (i, 0)),
            pl.BlockSpec((BR, OUT), lambda i: (i, 0)),
            pl.BlockSpec((1, OUT), lambda i: (0, 0)),
            pl.BlockSpec((OUT, 2), lambda i: (0, 0)),
            pl.BlockSpec((1, 2), lambda i: (0, 0)),
        ],
        out_specs=pl.BlockSpec((BR, 2), lambda i: (i, 0)),
        out_shape=jax.ShapeDtypeStruct((N, 2), _f32),
    )(p0, p1, b2, Wfc, bfc)


_MESH = plsc.VectorSubcoreMesh(
    core_axis_name="c", subcore_axis_name="s", num_cores=NC, num_subcores=NS)


def _leaky_exp(t):
    return jnp.exp(jnp.maximum(t, 0.2 * t))


@functools.partial(
    pl.kernel,
    out_type=(
        jax.ShapeDtypeStruct((E, HP), _f32),
        jax.ShapeDtypeStruct((NC, N, HP), _f32),
    ),
    mesh=_MESH,
    compiler_params=pltpu.CompilerParams(use_tc_tiling_on_sc=False, needs_layout_passes=False),
    scratch_types=[
        pltpu.VMEM_SHARED((N, HP), _f32),
        pltpu.VMEM_SHARED((N, HP), _f32),
        pltpu.VMEM_SHARED((N, HP), _f32),
        pltpu.VMEM((CH,), jnp.int32),
        pltpu.VMEM((CH,), jnp.int32),
        pltpu.VMEM((CH, HP), _f32),
        pltpu.VMEM((CH, HP), _f32),
        pltpu.SemaphoreType.DMA,
        pltpu.SemaphoreType.DMA,
    ],
)
def _passA1(src_hbm, dst_hbm, ast_hbm, adt_hbm, z16_hbm,
            e_hbm, dpart_hbm,
            as_sp, ad_sp, den_sp, src_v, dst_v, s_v, d_v, sem1, sem2):
    c = lax.axis_index("c")
    s = lax.axis_index("s")
    wid = c * NS + s

    @pl.when(s < N // ROWS_PER_STAGER)
    def _stage():
        rows = pl.ds(s * ROWS_PER_STAGER, ROWS_PER_STAGER)
        pltpu.sync_copy(ast_hbm.at[rows], as_sp.at[rows])
        pltpu.sync_copy(adt_hbm.at[rows], ad_sp.at[rows])
        pltpu.sync_copy(z16_hbm.at[rows], den_sp.at[rows])

    plsc.subcore_barrier()
    base = wid * EPW

    @pl.loop(0, NCHUNK)
    def _chunk(i):
        off = base + i * CH
        pltpu.sync_copy(src_hbm.at[pl.ds(off, CH)], src_v)
        pltpu.sync_copy(dst_hbm.at[pl.ds(off, CH)], dst_v)
        pltpu.async_copy(as_sp.at[src_v], s_v, sem1)
        pltpu.async_copy(ad_sp.at[dst_v], d_v, sem2)
        pltpu.make_async_copy(as_sp.at[src_v], s_v, sem1).wait()
        pltpu.make_async_copy(ad_sp.at[dst_v], d_v, sem2).wait()

        @pl.loop(0, CH)
        def _edge(j):
            t = s_v[j, :] + d_v[j, :]
            s_v[j, :] = _leaky_exp(t)

        pltpu.sync_copy(s_v, e_hbm.at[pl.ds(off, CH)])
        pltpu.sync_copy(s_v, den_sp.at[dst_v], add=True)

    plsc.subcore_barrier()

    @pl.when(s < N // ROWS_PER_STAGER)
    def _dump():
        rows = pl.ds(s * ROWS_PER_STAGER, ROWS_PER_STAGER)
        pltpu.sync_copy(den_sp.at[rows], dpart_hbm.at[c, rows])


@functools.partial(
    pl.kernel,
    out_type=jax.ShapeDtypeStruct((NC, N, HC1), _f32),
    mesh=_MESH,
    compiler_params=pltpu.CompilerParams(use_tc_tiling_on_sc=False, needs_layout_passes=False),
    scratch_types=[
        pltpu.VMEM_SHARED((N, HP), _f32),
        pltpu.VMEM_SHARED((N, HC1), _f32),
        pltpu.VMEM((ROWS_PER_STAGER, HP), _f32),
        pltpu.VMEM((ROWS_PER_STAGER, HP), _f32),
        pltpu.VMEM((CH,), jnp.int32),
        pltpu.VMEM((CH,), jnp.int32),
        pltpu.VMEM((CH, HP), _f32),
        pltpu.VMEM((CH, HP), _f32),
        pltpu.VMEM((CH, HC1), _f32),
        pltpu.SemaphoreType.DMA,
        pltpu.SemaphoreType.DMA,
    ],
)
def _passB1(src_hbm, dst_hbm, e_hbm, dp_hbm, h_hbm, z128_hbm,
            opart_hbm,
            den_sp, acc_sp, t0_v, t1_v, src_v, dst_v, e_v, den_v, h_v,
            sem1, sem2):
    c = lax.axis_index("c")
    s = lax.axis_index("s")
    wid = c * NS + s

    @pl.when(s < N // ROWS_PER_STAGER)
    def _stage():
        rows = pl.ds(s * ROWS_PER_STAGER, ROWS_PER_STAGER)
        pltpu.sync_copy(dp_hbm.at[0, rows], t0_v)
        pltpu.sync_copy(dp_hbm.at[1, rows], t1_v)

        @pl.loop(0, ROWS_PER_STAGER)
        def _sum(r):
            t0_v[r, :] = t0_v[r, :] + t1_v[r, :]

        pltpu.sync_copy(t0_v, den_sp.at[rows])
        pltpu.sync_copy(z128_hbm.at[rows], acc_sp.at[rows])

    plsc.subcore_barrier()
    base = wid * EPW

    @pl.loop(0, NCHUNK)
    def _chunk(i):
        off = base + i * CH
        pltpu.sync_copy(src_hbm.at[pl.ds(off, CH)], src_v)
        pltpu.sync_copy(dst_hbm.at[pl.ds(off, CH)], dst_v)
        pltpu.sync_copy(e_hbm.at[pl.ds(off, CH)], e_v)
        pltpu.async_copy(den_sp.at[dst_v], den_v, sem1)
        pltpu.async_copy(h_hbm.at[src_v], h_v, sem2)
        pltpu.make_async_copy(den_sp.at[dst_v], den_v, sem1).wait()
        pltpu.make_async_copy(h_hbm.at[src_v], h_v, sem2).wait()

        @pl.loop(0, CH)
        def _edge(j):
            w = e_v[j, :] / (den_v[j, :] + 1e-16)
            e_v[j, :] = w

        @pl.loop(0, CH)
        def _scale(j):
            wv = e_v[j, :]
            for h in range(HEADS):
                cols = pl.ds(h * HID, HID)
                h_v[j, cols] = h_v[j, cols] * wv[h]

        pltpu.sync_copy(h_v, acc_sp.at[dst_v], add=True)

    plsc.subcore_barrier()

    @pl.when(s < N // ROWS_PER_STAGER)
    def _dump():
        rows = pl.ds(s * ROWS_PER_STAGER, ROWS_PER_STAGER)
        pltpu.sync_copy(acc_sp.at[rows], opart_hbm.at[c, rows])


@functools.partial(
    pl.kernel,
    out_type=(
        jax.ShapeDtypeStruct((E,), _f32),
        jax.ShapeDtypeStruct((NC, N), _f32),
    ),
    mesh=_MESH,
    compiler_params=pltpu.CompilerParams(use_tc_tiling_on_sc=False, needs_layout_passes=False),
    scratch_types=[
        pltpu.VMEM_SHARED((N,), _f32),
        pltpu.VMEM((N,), _f32),
        pltpu.VMEM((N,), _f32),
        pltpu.VMEM((CH2,), jnp.int32),
        pltpu.VMEM((CH2,), jnp.int32),
        pltpu.VMEM((CH2,), _f32),
    ],
)
def _passA2(src_hbm, dst_hbm, ast_hbm, adt_hbm, z1_hbm,
            e_hbm, dpart_hbm,
            den_sp, as_v, ad_v, src_v, dst_v, e_v):
    c = lax.axis_index("c")
    s = lax.axis_index("s")
    wid = c * NS + s

    pltpu.sync_copy(ast_hbm, as_v)
    pltpu.sync_copy(adt_hbm, ad_v)

    @pl.when(s < N // ROWS_PER_STAGER)
    def _zero():
        rows = pl.ds(s * ROWS_PER_STAGER, ROWS_PER_STAGER)
        pltpu.sync_copy(z1_hbm.at[rows], den_sp.at[rows])

    plsc.subcore_barrier()
    base = wid * EPW

    @pl.loop(0, NCHUNK2)
    def _chunk(i):
        off = base + i * CH2
        pltpu.sync_copy(src_hbm.at[pl.ds(off, CH2)], src_v)
        pltpu.sync_copy(dst_hbm.at[pl.ds(off, CH2)], dst_v)

        for k in range(CH2 // 16):
            sl = pl.ds(k * 16, 16)
            si = src_v[sl]
            di = dst_v[sl]
            sv = plsc.load_gather(as_v, [si])
            dv = plsc.load_gather(ad_v, [di])
            e_v[sl] = _leaky_exp(sv + dv)

        pltpu.sync_copy(e_v, e_hbm.at[pl.ds(off, CH2)])
        pltpu.sync_copy(e_v, den_sp.at[dst_v], add=True)

    plsc.subcore_barrier()

    @pl.when(s < N // ROWS_PER_STAGER)
    def _dump():
        rows = pl.ds(s * ROWS_PER_STAGER, ROWS_PER_STAGER)
        pltpu.sync_copy(den_sp.at[rows], dpart_hbm.at[c, rows])


@functools.partial(
    pl.kernel,
    out_type=jax.ShapeDtypeStruct((NC, N, OUT), _f32),
    mesh=_MESH,
    compiler_params=pltpu.CompilerParams(use_tc_tiling_on_sc=False, needs_layout_passes=False),
    scratch_types=[
        pltpu.VMEM_SHARED((N, OUT), _f32),
        pltpu.VMEM((N,), _f32),
        pltpu.VMEM((N,), _f32),
        pltpu.VMEM((CH2,), jnp.int32),
        pltpu.VMEM((CH2,), jnp.int32),
        pltpu.VMEM((CH2,), _f32),
        pltpu.VMEM((CH2, OUT), _f32),
        pltpu.SemaphoreType.DMA,
    ],
)
def _passB2(src_hbm, dst_hbm, e_hbm, dp_hbm, h2_hbm, z64_hbm,
            opart_hbm,
            acc_sp, den_v, tmp_v, src_v, dst_v, e_v, h_v, sem1):
    c = lax.axis_index("c")
    s = lax.axis_index("s")
    wid = c * NS + s

    pltpu.sync_copy(dp_hbm.at[0], den_v)
    pltpu.sync_copy(dp_hbm.at[1], tmp_v)

    @pl.loop(0, N // 16)
    def _sumden(r):
        sl = pl.ds(r * 16, 16)
        den_v[sl] = den_v[sl] + tmp_v[sl] + 1e-16

    @pl.when(s < N // ROWS_PER_STAGER)
    def _zero():
        rows = pl.ds(s * ROWS_PER_STAGER, ROWS_PER_STAGER)
        pltpu.sync_copy(z64_hbm.at[rows], acc_sp.at[rows])

    plsc.subcore_barrier()
    base = wid * EPW

    @pl.loop(0, NCHUNK2)
    def _chunk(i):
        off = base + i * CH2
        pltpu.sync_copy(src_hbm.at[pl.ds(off, CH2)], src_v)
        pltpu.sync_copy(dst_hbm.at[pl.ds(off, CH2)], dst_v)
        pltpu.sync_copy(e_hbm.at[pl.ds(off, CH2)], e_v)
        pltpu.async_copy(h2_hbm.at[src_v], h_v, sem1)

        for k in range(CH2 // 16):
            sl = pl.ds(k * 16, 16)
            di = dst_v[sl]
            dv = plsc.load_gather(den_v, [di])
            e_v[sl] = e_v[sl] / dv

        pltpu.make_async_copy(h2_hbm.at[src_v], h_v, sem1).wait()

        @pl.loop(0, CH2 // 16)
        def _scale(k):
            wv = e_v[pl.ds(k * 16, 16)]
            for jj in range(16):
                w = wv[jj]
                for q in range(OUT // 16):
                    cols = pl.ds(q * 16, 16)
                    h_v[k * 16 + jj, cols] = h_v[k * 16 + jj, cols] * w

        pltpu.sync_copy(h_v, acc_sp.at[dst_v], add=True)

    plsc.subcore_barrier()

    @pl.when(s < N // ROWS_PER_STAGER)
    def _dump():
        rows = pl.ds(s * ROWS_PER_STAGER, ROWS_PER_STAGER)
        pltpu.sync_copy(acc_sp.at[rows], opart_hbm.at[c, rows])


def kernel(x, edge_index, W1, a_src1, a_dst1, b1, W2, a_src2, a_dst2, b2,
           Wfc, bfc):
    src = edge_index[0]
    dst = edge_index[1]

    j = jnp.arange(HC1)
    A1s = jnp.zeros((HC1, HP), _f32).at[j, j // HID].set(a_src1.reshape(-1))
    A1d = jnp.zeros((HC1, HP), _f32).at[j, j // HID].set(a_dst1.reshape(-1))
    avs2 = a_src2.reshape(OUT, 1)
    avd2 = a_dst2.reshape(OUT, 1)

    z16 = jnp.zeros((N, HP), _f32)
    z128 = jnp.zeros((N, HC1), _f32)
    z64 = jnp.zeros((N, OUT), _f32)
    z1 = jnp.zeros((N,), _f32)

    h1, as1, ad1 = _dense1(x, W1, A1s, A1d)
    e1, dp1 = _passA1(src, dst, as1, ad1, z16)
    op1 = _passB1(src, dst, e1, dp1, h1, z128)

    h2, as2, ad2 = _dense2(op1[0], op1[1], b1.reshape(1, HC1), W2, avs2, avd2)
    e2, dp2 = _passA2(src, dst, as2.reshape(N), ad2.reshape(N), z1)
    op2 = _passB2(src, dst, e2, dp2, h2, z64)

    return _final(op2[0], op2[1], b2.reshape(1, OUT), Wfc, bfc.reshape(1, 2))


# speedup vs baseline: 61.4946x; 2.2590x over previous
import functools

import jax
import jax.numpy as jnp
from jax import lax
from jax.experimental import pallas as pl
from jax.experimental.pallas import tpu as pltpu
from jax.experimental.pallas import tpu_sc as plsc

N = 10000
E = 320000
IN = 128
HID = 16
HEADS = 8
OUT = 64
HC1 = HEADS * HID
HP = 16

NC = 2
NS = 16
NW = NC * NS
EPW = E // NW
CH1 = 40
NCHUNK1 = EPW // CH1
CH2 = 80
NCHUNK2 = EPW // CH2
ROWS_PER_STAGER = 1000

_f32 = jnp.float32


BR = 400


def _dense1_body(x_ref, w_ref, asrc_ref, adst_ref, h_ref, as_ref, ad_ref):
    h = jnp.dot(x_ref[...], w_ref[...], preferred_element_type=_f32)
    h_ref[...] = h
    as_ref[...] = jnp.dot(h, asrc_ref[...], preferred_element_type=_f32,
                          precision=lax.Precision.HIGHEST)
    ad_ref[...] = jnp.dot(h, adst_ref[...], preferred_element_type=_f32,
                          precision=lax.Precision.HIGHEST)


def _dense1(x, W1, A1s, A1d):
    return pl.pallas_call(
        _dense1_body,
        grid=(N // BR,),
        in_specs=[
            pl.BlockSpec((BR, IN), lambda i: (i, 0)),
            pl.BlockSpec((IN, HC1), lambda i: (0, 0)),
            pl.BlockSpec((HC1, HP), lambda i: (0, 0)),
            pl.BlockSpec((HC1, HP), lambda i: (0, 0)),
        ],
        out_specs=[
            pl.BlockSpec((BR, HC1), lambda i: (i, 0)),
            pl.BlockSpec((BR, HP), lambda i: (i, 0)),
            pl.BlockSpec((BR, HP), lambda i: (i, 0)),
        ],
        out_shape=[
            jax.ShapeDtypeStruct((N, HC1), _f32),
            jax.ShapeDtypeStruct((N, HP), _f32),
            jax.ShapeDtypeStruct((N, HP), _f32),
        ],
    )(x, W1, A1s, A1d)


def _dense2_body(p0_ref, p1_ref, d0_ref, d1_ref, exp_ref, b1_ref, w2_ref,
                 avs_ref, avd_ref, h2_ref, as_ref, ad_ref):
    recip = 1.0 / (d0_ref[...] + d1_ref[...] + 1e-16)
    rep = jnp.dot(recip, exp_ref[...], preferred_element_type=_f32,
                  precision=lax.Precision.HIGHEST)
    x2 = jnp.maximum((p0_ref[...] + p1_ref[...]) * rep + b1_ref[...], 0.0)
    h2 = jnp.dot(x2, w2_ref[...], preferred_element_type=_f32)
    h2_ref[...] = h2
    as_ref[...] = jnp.dot(h2, avs_ref[...], preferred_element_type=_f32,
                          precision=lax.Precision.HIGHEST)
    ad_ref[...] = jnp.dot(h2, avd_ref[...], preferred_element_type=_f32,
                          precision=lax.Precision.HIGHEST)


def _dense2(p0, p1, d0, d1, EXPAND, b1, W2, avs, avd):
    return pl.pallas_call(
        _dense2_body,
        grid=(N // BR,),
        in_specs=[
            pl.BlockSpec((BR, HC1), lambda i: (i, 0)),
            pl.BlockSpec((BR, HC1), lambda i: (i, 0)),
            pl.BlockSpec((BR, HP), lambda i: (i, 0)),
            pl.BlockSpec((BR, HP), lambda i: (i, 0)),
            pl.BlockSpec((HP, HC1), lambda i: (0, 0)),
            pl.BlockSpec((1, HC1), lambda i: (0, 0)),
            pl.BlockSpec((HC1, OUT), lambda i: (0, 0)),
            pl.BlockSpec((OUT, 1), lambda i: (0, 0)),
            pl.BlockSpec((OUT, 1), lambda i: (0, 0)),
        ],
        out_specs=[
            pl.BlockSpec((BR, OUT), lambda i: (i, 0)),
            pl.BlockSpec((BR, 1), lambda i: (i, 0)),
            pl.BlockSpec((BR, 1), lambda i: (i, 0)),
        ],
        out_shape=[
            jax.ShapeDtypeStruct((N, OUT), _f32),
            jax.ShapeDtypeStruct((N, 1), _f32),
            jax.ShapeDtypeStruct((N, 1), _f32),
        ],
    )(p0, p1, d0, d1, EXPAND, b1, W2, avs, avd)


def _final_body(p0_ref, p1_ref, d0_ref, d1_ref, b2_ref, wfc_ref, bfc_ref,
                out_ref):
    recip = 1.0 / (d0_ref[...] + d1_ref[...] + 1e-16)
    y = (p0_ref[...] + p1_ref[...]) * recip + b2_ref[...]
    out_ref[...] = jnp.dot(y, wfc_ref[...], preferred_element_type=_f32) + bfc_ref[...]


def _final(p0, p1, d0, d1, b2, Wfc, bfc):
    return pl.pallas_call(
        _final_body,
        grid=(N // BR,),
        in_specs=[
            pl.BlockSpec((BR, OUT), lambda i: (i, 0)),
            pl.BlockSpec((BR, OUT), lambda i: (i, 0)),
            pl.BlockSpec((BR, 1), lambda i: (i, 0)),
            pl.BlockSpec((BR, 1), lambda i: (i, 0)),
            pl.BlockSpec((1, OUT), lambda i: (0, 0)),
            pl.BlockSpec((OUT, 2), lambda i: (0, 0)),
            pl.BlockSpec((1, 2), lambda i: (0, 0)),
        ],
        out_specs=pl.BlockSpec((BR, 2), lambda i: (i, 0)),
        out_shape=jax.ShapeDtypeStruct((N, 2), _f32),
    )(p0, p1, d0, d1, b2, Wfc, bfc)


_MESH = plsc.VectorSubcoreMesh(
    core_axis_name="c", subcore_axis_name="s", num_cores=NC, num_subcores=NS)


def _leaky_exp(t):
    return jnp.exp(jnp.maximum(t, 0.2 * t))


@functools.partial(
    pl.kernel,
    out_type=(
        jax.ShapeDtypeStruct((NC, N, HP), _f32),
        jax.ShapeDtypeStruct((NC, N, HC1), _f32),
    ),
    mesh=_MESH,
    compiler_params=pltpu.CompilerParams(use_tc_tiling_on_sc=False, needs_layout_passes=False),
    scratch_types=[
        pltpu.VMEM_SHARED((N, HP), _f32),
        pltpu.VMEM_SHARED((N, HC1), _f32),
        pltpu.VMEM((EPW,), jnp.int32),
        pltpu.VMEM((EPW,), jnp.int32),
        pltpu.VMEM((CH1, HP), _f32),
        pltpu.VMEM((CH1, HP), _f32),
        pltpu.VMEM((CH1, HP), _f32),
        pltpu.VMEM((CH1, HP), _f32),
        pltpu.VMEM((CH1, HC1), _f32),
        pltpu.VMEM((CH1, HC1), _f32),
        pltpu.SemaphoreType.DMA,
        pltpu.SemaphoreType.DMA,
        pltpu.SemaphoreType.DMA,
        pltpu.SemaphoreType.DMA,
        pltpu.SemaphoreType.DMA,
        pltpu.SemaphoreType.DMA,
    ],
)
def _edge1(src_hbm, dst_hbm, ast_hbm, adt_hbm, h_hbm, z16_hbm, z128_hbm,
           dpart_hbm, opart_hbm,
           den_sp, acc_sp, src_all, dst_all,
           s_v0, s_v1, d_v0, d_v1, h_v0, h_v1,
           semA0, semA1, semB0, semB1, semH0, semH1):
    c = lax.axis_index("c")
    s = lax.axis_index("s")
    wid = c * NS + s
    base = wid * EPW
    sv = (s_v0, s_v1)
    dv = (d_v0, d_v1)
    hv = (h_v0, h_v1)
    semA = (semA0, semA1)
    semB = (semB0, semB1)
    semH = (semH0, semH1)

    @pl.when(s < N // ROWS_PER_STAGER)
    def _zero():
        rows = pl.ds(s * ROWS_PER_STAGER, ROWS_PER_STAGER)
        pltpu.sync_copy(z16_hbm.at[rows], den_sp.at[rows])
        pltpu.sync_copy(z128_hbm.at[rows], acc_sp.at[rows])

    pltpu.sync_copy(src_hbm.at[pl.ds(base, EPW)], src_all)
    pltpu.sync_copy(dst_hbm.at[pl.ds(base, EPW)], dst_all)
    plsc.subcore_barrier()

    def _start(i, b):
        sl = pl.ds(i * CH1, CH1)
        pltpu.async_copy(ast_hbm.at[src_all.at[sl]], sv[b], semA[b])
        pltpu.async_copy(adt_hbm.at[dst_all.at[sl]], dv[b], semB[b])
        pltpu.async_copy(h_hbm.at[src_all.at[sl]], hv[b], semH[b])

    def _finish(i, b):
        sl = pl.ds(i * CH1, CH1)
        pltpu.make_async_copy(ast_hbm.at[src_all.at[sl]], sv[b], semA[b]).wait()
        pltpu.make_async_copy(adt_hbm.at[dst_all.at[sl]], dv[b], semB[b]).wait()
        pltpu.make_async_copy(h_hbm.at[src_all.at[sl]], hv[b], semH[b]).wait()

        @pl.loop(0, CH1)
        def _edge(j):
            ev = _leaky_exp(sv[b][j, :] + dv[b][j, :])
            sv[b][j, :] = ev
            for h in range(HEADS):
                cols = pl.ds(h * HID, HID)
                hv[b][j, cols] = hv[b][j, cols] * ev[h]

        pltpu.sync_copy(sv[b], den_sp.at[dst_all.at[sl]], add=True)
        pltpu.sync_copy(hv[b], acc_sp.at[dst_all.at[sl]], add=True)

    _start(0, 0)
    _start(1, 1)

    @pl.loop(0, (NCHUNK1 - 2) // 2)
    def _pair(k):
        i = k * 2
        _finish(i, 0)
        _start(i + 2, 0)
        _finish(i + 1, 1)

        @pl.when(i + 3 < NCHUNK1)
        def _():
            _start(i + 3, 1)

    _finish(NCHUNK1 - 2, 0)
    _finish(NCHUNK1 - 1, 1)
    plsc.subcore_barrier()

    @pl.when(s < N // ROWS_PER_STAGER)
    def _dump():
        rows = pl.ds(s * ROWS_PER_STAGER, ROWS_PER_STAGER)
        pltpu.sync_copy(den_sp.at[rows], dpart_hbm.at[c, rows])
        pltpu.sync_copy(acc_sp.at[rows], opart_hbm.at[c, rows])


@functools.partial(
    pl.kernel,
    out_type=(
        jax.ShapeDtypeStruct((NC, N), _f32),
        jax.ShapeDtypeStruct((NC, N, OUT), _f32),
    ),
    mesh=_MESH,
    compiler_params=pltpu.CompilerParams(use_tc_tiling_on_sc=False, needs_layout_passes=False),
    scratch_types=[
        pltpu.VMEM_SHARED((N,), _f32),
        pltpu.VMEM_SHARED((N, OUT), _f32),
        pltpu.VMEM((N,), _f32),
        pltpu.VMEM((N,), _f32),
        pltpu.VMEM((CH2,), jnp.int32),
        pltpu.VMEM((CH2,), jnp.int32),
        pltpu.VMEM((CH2,), _f32),
        pltpu.VMEM((CH2, OUT), _f32),
        pltpu.SemaphoreType.DMA,
    ],
)
def _edge2(src_hbm, dst_hbm, ast_hbm, adt_hbm, h2_hbm, z1_hbm, z64_hbm,
           dpart_hbm, opart_hbm,
           den_sp, acc_sp, as_v, ad_v, src_v, dst_v, e_v, h_v, sem1):
    c = lax.axis_index("c")
    s = lax.axis_index("s")
    wid = c * NS + s

    pltpu.sync_copy(ast_hbm, as_v)
    pltpu.sync_copy(adt_hbm, ad_v)

    @pl.when(s < N // ROWS_PER_STAGER)
    def _zero():
        rows = pl.ds(s * ROWS_PER_STAGER, ROWS_PER_STAGER)
        pltpu.sync_copy(z1_hbm.at[rows], den_sp.at[rows])
        pltpu.sync_copy(z64_hbm.at[rows], acc_sp.at[rows])

    plsc.subcore_barrier()
    base = wid * EPW

    @pl.loop(0, NCHUNK2)
    def _chunk(i):
        off = base + i * CH2
        pltpu.sync_copy(src_hbm.at[pl.ds(off, CH2)], src_v)
        pltpu.sync_copy(dst_hbm.at[pl.ds(off, CH2)], dst_v)
        pltpu.async_copy(h2_hbm.at[src_v], h_v, sem1)

        for k in range(CH2 // 16):
            sl = pl.ds(k * 16, 16)
            si = src_v[sl]
            di = dst_v[sl]
            svv = plsc.load_gather(as_v, [si])
            dvv = plsc.load_gather(ad_v, [di])
            e_v[sl] = _leaky_exp(svv + dvv)

        pltpu.sync_copy(e_v, den_sp.at[dst_v], add=True)
        pltpu.make_async_copy(h2_hbm.at[src_v], h_v, sem1).wait()

        @pl.loop(0, CH2 // 16)
        def _scale(k):
            wv = e_v[pl.ds(k * 16, 16)]
            for jj in range(16):
                w = wv[jj]
                for q in range(OUT // 16):
                    cols = pl.ds(q * 16, 16)
                    h_v[k * 16 + jj, cols] = h_v[k * 16 + jj, cols] * w

        pltpu.sync_copy(h_v, acc_sp.at[dst_v], add=True)

    plsc.subcore_barrier()

    @pl.when(s < N // ROWS_PER_STAGER)
    def _dump():
        rows = pl.ds(s * ROWS_PER_STAGER, ROWS_PER_STAGER)
        pltpu.sync_copy(den_sp.at[rows], dpart_hbm.at[c, rows])
        pltpu.sync_copy(acc_sp.at[rows], opart_hbm.at[c, rows])


def kernel(x, edge_index, W1, a_src1, a_dst1, b1, W2, a_src2, a_dst2, b2,
           Wfc, bfc):
    src = edge_index[0]
    dst = edge_index[1]

    j = jnp.arange(HC1)
    A1s = jnp.zeros((HC1, HP), _f32).at[j, j // HID].set(a_src1.reshape(-1))
    A1d = jnp.zeros((HC1, HP), _f32).at[j, j // HID].set(a_dst1.reshape(-1))
    avs2 = a_src2.reshape(OUT, 1)
    avd2 = a_dst2.reshape(OUT, 1)
    EXP = jnp.zeros((HP, HC1), _f32).at[j // HID, j].set(1.0)

    z16 = jnp.zeros((N, HP), _f32)
    z128 = jnp.zeros((N, HC1), _f32)
    z64 = jnp.zeros((N, OUT), _f32)
    z1 = jnp.zeros((N,), _f32)

    h1, as1, ad1 = _dense1(x, W1, A1s, A1d)
    dp1, op1 = _edge1(src, dst, as1, ad1, h1, z16, z128)

    h2, as2, ad2 = _dense2(op1[0], op1[1], dp1[0], dp1[1], EXP,
                           b1.reshape(1, HC1), W2, avs2, avd2)
    dp2, op2 = _edge2(src, dst, as2.reshape(N), ad2.reshape(N), h2, z1, z64)

    return _final(op2[0], op2[1], dp2[0].reshape(N, 1), dp2[1].reshape(N, 1),
                  b2.reshape(1, OUT), Wfc, bfc.reshape(1, 2))


# speedup vs baseline: 65.8864x; 1.0714x over previous
import functools

import jax
import jax.numpy as jnp
from jax import lax
from jax.experimental import pallas as pl
from jax.experimental.pallas import tpu as pltpu
from jax.experimental.pallas import tpu_sc as plsc

N = 10000
E = 320000
IN = 128
HID = 16
HEADS = 8
OUT = 64
HC1 = HEADS * HID
HP = 16

NC = 2
NS = 16
NW = NC * NS
EPW = E // NW
CH1 = 40
NCHUNK1 = EPW // CH1
CH2 = 80
NCHUNK2 = EPW // CH2
ROWS_PER_STAGER = 1000

_f32 = jnp.float32


BR = 400


def _dense1_body(x_ref, w_ref, asrc_ref, adst_ref, h_ref, as_ref, ad_ref):
    h = jnp.dot(x_ref[...], w_ref[...], preferred_element_type=_f32)
    h_ref[...] = h
    as_ref[...] = jnp.dot(h, asrc_ref[...], preferred_element_type=_f32,
                          precision=lax.Precision.HIGHEST)
    ad_ref[...] = jnp.dot(h, adst_ref[...], preferred_element_type=_f32,
                          precision=lax.Precision.HIGHEST)


def _dense1(x, W1, A1s, A1d):
    return pl.pallas_call(
        _dense1_body,
        grid=(N // BR,),
        in_specs=[
            pl.BlockSpec((BR, IN), lambda i: (i, 0)),
            pl.BlockSpec((IN, HC1), lambda i: (0, 0)),
            pl.BlockSpec((HC1, HP), lambda i: (0, 0)),
            pl.BlockSpec((HC1, HP), lambda i: (0, 0)),
        ],
        out_specs=[
            pl.BlockSpec((BR, HC1), lambda i: (i, 0)),
            pl.BlockSpec((BR, HP), lambda i: (i, 0)),
            pl.BlockSpec((BR, HP), lambda i: (i, 0)),
        ],
        out_shape=[
            jax.ShapeDtypeStruct((N, HC1), _f32),
            jax.ShapeDtypeStruct((N, HP), _f32),
            jax.ShapeDtypeStruct((N, HP), _f32),
        ],
    )(x, W1, A1s, A1d)


def _dense2_body(p0_ref, p1_ref, d0_ref, d1_ref, exp_ref, b1_ref, w2_ref,
                 avs_ref, avd_ref, h2_ref, as_ref, ad_ref):
    recip = 1.0 / (d0_ref[...] + d1_ref[...] + 1e-16)
    rep = jnp.dot(recip, exp_ref[...], preferred_element_type=_f32,
                  precision=lax.Precision.HIGHEST)
    x2 = jnp.maximum((p0_ref[...] + p1_ref[...]) * rep + b1_ref[...], 0.0)
    h2 = jnp.dot(x2, w2_ref[...], preferred_element_type=_f32)
    h2_ref[...] = h2
    as_ref[...] = jnp.dot(h2, avs_ref[...], preferred_element_type=_f32,
                          precision=lax.Precision.HIGHEST)
    ad_ref[...] = jnp.dot(h2, avd_ref[...], preferred_element_type=_f32,
                          precision=lax.Precision.HIGHEST)


def _dense2(p0, p1, d0, d1, EXPAND, b1, W2, avs, avd):
    return pl.pallas_call(
        _dense2_body,
        grid=(N // BR,),
        in_specs=[
            pl.BlockSpec((BR, HC1), lambda i: (i, 0)),
            pl.BlockSpec((BR, HC1), lambda i: (i, 0)),
            pl.BlockSpec((BR, HP), lambda i: (i, 0)),
            pl.BlockSpec((BR, HP), lambda i: (i, 0)),
            pl.BlockSpec((HP, HC1), lambda i: (0, 0)),
            pl.BlockSpec((1, HC1), lambda i: (0, 0)),
            pl.BlockSpec((HC1, OUT), lambda i: (0, 0)),
            pl.BlockSpec((OUT, 1), lambda i: (0, 0)),
            pl.BlockSpec((OUT, 1), lambda i: (0, 0)),
        ],
        out_specs=[
            pl.BlockSpec((BR, OUT), lambda i: (i, 0)),
            pl.BlockSpec((BR, 1), lambda i: (i, 0)),
            pl.BlockSpec((BR, 1), lambda i: (i, 0)),
        ],
        out_shape=[
            jax.ShapeDtypeStruct((N, OUT), _f32),
            jax.ShapeDtypeStruct((N, 1), _f32),
            jax.ShapeDtypeStruct((N, 1), _f32),
        ],
    )(p0, p1, d0, d1, EXPAND, b1, W2, avs, avd)


def _final_body(p0_ref, p1_ref, d0_ref, d1_ref, b2_ref, wfc_ref, bfc_ref,
                out_ref):
    recip = 1.0 / (d0_ref[...] + d1_ref[...] + 1e-16)
    y = (p0_ref[...] + p1_ref[...]) * recip + b2_ref[...]
    out_ref[...] = jnp.dot(y, wfc_ref[...], preferred_element_type=_f32) + bfc_ref[...]


def _final(p0, p1, d0, d1, b2, Wfc, bfc):
    return pl.pallas_call(
        _final_body,
        grid=(N // BR,),
        in_specs=[
            pl.BlockSpec((BR, OUT), lambda i: (i, 0)),
            pl.BlockSpec((BR, OUT), lambda i: (i, 0)),
            pl.BlockSpec((BR, 1), lambda i: (i, 0)),
            pl.BlockSpec((BR, 1), lambda i: (i, 0)),
            pl.BlockSpec((1, OUT), lambda i: (0, 0)),
            pl.BlockSpec((OUT, 2), lambda i: (0, 0)),
            pl.BlockSpec((1, 2), lambda i: (0, 0)),
        ],
        out_specs=pl.BlockSpec((BR, 2), lambda i: (i, 0)),
        out_shape=jax.ShapeDtypeStruct((N, 2), _f32),
    )(p0, p1, d0, d1, b2, Wfc, bfc)


_MESH = plsc.VectorSubcoreMesh(
    core_axis_name="c", subcore_axis_name="s", num_cores=NC, num_subcores=NS)


def _leaky_exp(t):
    return jnp.exp(jnp.maximum(t, 0.2 * t))


@functools.partial(
    pl.kernel,
    out_type=(
        jax.ShapeDtypeStruct((NC, N, HP), _f32),
        jax.ShapeDtypeStruct((NC, N, HC1), _f32),
    ),
    mesh=_MESH,
    compiler_params=pltpu.CompilerParams(use_tc_tiling_on_sc=False, needs_layout_passes=False),
    scratch_types=[
        pltpu.VMEM_SHARED((N, HP), _f32),
        pltpu.VMEM_SHARED((N, HC1), _f32),
        pltpu.VMEM((EPW,), jnp.int32),
        pltpu.VMEM((EPW,), jnp.int32),
        pltpu.VMEM((CH1, HP), _f32),
        pltpu.VMEM((CH1, HP), _f32),
        pltpu.VMEM((CH1, HP), _f32),
        pltpu.VMEM((CH1, HP), _f32),
        pltpu.VMEM((CH1, HC1), _f32),
        pltpu.VMEM((CH1, HC1), _f32),
        pltpu.SemaphoreType.DMA,
        pltpu.SemaphoreType.DMA,
        pltpu.SemaphoreType.DMA,
        pltpu.SemaphoreType.DMA,
        pltpu.SemaphoreType.DMA,
        pltpu.SemaphoreType.DMA,
    ],
)
def _edge1(src_hbm, dst_hbm, ast_hbm, adt_hbm, h_hbm, z16_hbm, z128_hbm,
           dpart_hbm, opart_hbm,
           den_sp, acc_sp, src_all, dst_all,
           s_v0, s_v1, d_v0, d_v1, h_v0, h_v1,
           semA0, semA1, semB0, semB1, semH0, semH1):
    c = lax.axis_index("c")
    s = lax.axis_index("s")
    wid = c * NS + s
    base = wid * EPW
    sv = (s_v0, s_v1)
    dv = (d_v0, d_v1)
    hv = (h_v0, h_v1)
    semA = (semA0, semA1)
    semB = (semB0, semB1)
    semH = (semH0, semH1)

    @pl.when(s < N // ROWS_PER_STAGER)
    def _zero():
        rows = pl.ds(s * ROWS_PER_STAGER, ROWS_PER_STAGER)
        pltpu.sync_copy(z16_hbm.at[rows], den_sp.at[rows])
        pltpu.sync_copy(z128_hbm.at[rows], acc_sp.at[rows])

    pltpu.sync_copy(src_hbm.at[pl.ds(base, EPW)], src_all)
    pltpu.sync_copy(dst_hbm.at[pl.ds(base, EPW)], dst_all)
    plsc.subcore_barrier()

    def _start(i, b):
        sl = pl.ds(i * CH1, CH1)
        pltpu.async_copy(ast_hbm.at[src_all.at[sl]], sv[b], semA[b])
        pltpu.async_copy(adt_hbm.at[dst_all.at[sl]], dv[b], semB[b])
        pltpu.async_copy(h_hbm.at[src_all.at[sl]], hv[b], semH[b])

    def _finish(i, b):
        sl = pl.ds(i * CH1, CH1)
        pltpu.make_async_copy(ast_hbm.at[src_all.at[sl]], sv[b], semA[b]).wait()
        pltpu.make_async_copy(adt_hbm.at[dst_all.at[sl]], dv[b], semB[b]).wait()
        pltpu.make_async_copy(h_hbm.at[src_all.at[sl]], hv[b], semH[b]).wait()

        @pl.loop(0, CH1)
        def _edge(j):
            ev = _leaky_exp(sv[b][j, :] + dv[b][j, :])
            sv[b][j, :] = ev
            for h in range(HEADS):
                cols = pl.ds(h * HID, HID)
                hv[b][j, cols] = hv[b][j, cols] * ev[h]

        pltpu.sync_copy(sv[b], den_sp.at[dst_all.at[sl]], add=True)
        pltpu.sync_copy(hv[b], acc_sp.at[dst_all.at[sl]], add=True)

    _start(0, 0)
    _start(1, 1)

    @pl.loop(0, (NCHUNK1 - 2) // 2)
    def _pair(k):
        i = k * 2
        _finish(i, 0)
        _start(i + 2, 0)
        _finish(i + 1, 1)

        @pl.when(i + 3 < NCHUNK1)
        def _():
            _start(i + 3, 1)

    _finish(NCHUNK1 - 2, 0)
    _finish(NCHUNK1 - 1, 1)
    plsc.subcore_barrier()

    @pl.when(s < N // ROWS_PER_STAGER)
    def _dump():
        rows = pl.ds(s * ROWS_PER_STAGER, ROWS_PER_STAGER)
        pltpu.sync_copy(den_sp.at[rows], dpart_hbm.at[c, rows])
        pltpu.sync_copy(acc_sp.at[rows], opart_hbm.at[c, rows])


@functools.partial(
    pl.kernel,
    out_type=(
        jax.ShapeDtypeStruct((NC, N), _f32),
        jax.ShapeDtypeStruct((NC, N, OUT), _f32),
    ),
    mesh=_MESH,
    compiler_params=pltpu.CompilerParams(use_tc_tiling_on_sc=False, needs_layout_passes=False),
    scratch_types=[
        pltpu.VMEM_SHARED((N,), _f32),
        pltpu.VMEM_SHARED((N, OUT), _f32),
        pltpu.VMEM_SHARED((N, OUT), _f32),
        pltpu.VMEM((N,), _f32),
        pltpu.VMEM((N,), _f32),
        pltpu.VMEM((CH2,), jnp.int32),
        pltpu.VMEM((CH2,), jnp.int32),
        pltpu.VMEM((CH2,), _f32),
        pltpu.VMEM((CH2, OUT), _f32),
        pltpu.SemaphoreType.DMA,
    ],
)
def _edge2(src_hbm, dst_hbm, ast_hbm, adt_hbm, h2_hbm, z1_hbm, z64_hbm,
           dpart_hbm, opart_hbm,
           den_sp, acc_sp, h2_sp, as_v, ad_v, src_v, dst_v, e_v, h_v, sem1):
    c = lax.axis_index("c")
    s = lax.axis_index("s")
    wid = c * NS + s

    pltpu.sync_copy(ast_hbm, as_v)
    pltpu.sync_copy(adt_hbm, ad_v)

    @pl.when(s < N // ROWS_PER_STAGER)
    def _zero():
        rows = pl.ds(s * ROWS_PER_STAGER, ROWS_PER_STAGER)
        pltpu.sync_copy(z1_hbm.at[rows], den_sp.at[rows])
        pltpu.sync_copy(z64_hbm.at[rows], acc_sp.at[rows])
        pltpu.sync_copy(h2_hbm.at[rows], h2_sp.at[rows])

    plsc.subcore_barrier()
    base = wid * EPW

    @pl.loop(0, NCHUNK2)
    def _chunk(i):
        off = base + i * CH2
        pltpu.sync_copy(src_hbm.at[pl.ds(off, CH2)], src_v)
        pltpu.sync_copy(dst_hbm.at[pl.ds(off, CH2)], dst_v)
        pltpu.async_copy(h2_sp.at[src_v], h_v, sem1)

        for k in range(CH2 // 16):
            sl = pl.ds(k * 16, 16)
            si = src_v[sl]
            di = dst_v[sl]
            svv = plsc.load_gather(as_v, [si])
            dvv = plsc.load_gather(ad_v, [di])
            e_v[sl] = _leaky_exp(svv + dvv)

        pltpu.sync_copy(e_v, den_sp.at[dst_v], add=True)
        pltpu.make_async_copy(h2_sp.at[src_v], h_v, sem1).wait()

        @pl.loop(0, CH2 // 16)
        def _scale(k):
            wv = e_v[pl.ds(k * 16, 16)]
            for jj in range(16):
                w = wv[jj]
                for q in range(OUT // 16):
                    cols = pl.ds(q * 16, 16)
                    h_v[k * 16 + jj, cols] = h_v[k * 16 + jj, cols] * w

        pltpu.sync_copy(h_v, acc_sp.at[dst_v], add=True)

    plsc.subcore_barrier()

    @pl.when(s < N // ROWS_PER_STAGER)
    def _dump():
        rows = pl.ds(s * ROWS_PER_STAGER, ROWS_PER_STAGER)
        pltpu.sync_copy(den_sp.at[rows], dpart_hbm.at[c, rows])
        pltpu.sync_copy(acc_sp.at[rows], opart_hbm.at[c, rows])


def kernel(x, edge_index, W1, a_src1, a_dst1, b1, W2, a_src2, a_dst2, b2,
           Wfc, bfc):
    src = edge_index[0]
    dst = edge_index[1]

    j = jnp.arange(HC1)
    A1s = jnp.zeros((HC1, HP), _f32).at[j, j // HID].set(a_src1.reshape(-1))
    A1d = jnp.zeros((HC1, HP), _f32).at[j, j // HID].set(a_dst1.reshape(-1))
    avs2 = a_src2.reshape(OUT, 1)
    avd2 = a_dst2.reshape(OUT, 1)
    EXP = jnp.zeros((HP, HC1), _f32).at[j // HID, j].set(1.0)

    z16 = jnp.zeros((N, HP), _f32)
    z128 = jnp.zeros((N, HC1), _f32)
    z64 = jnp.zeros((N, OUT), _f32)
    z1 = jnp.zeros((N,), _f32)

    h1, as1, ad1 = _dense1(x, W1, A1s, A1d)
    dp1, op1 = _edge1(src, dst, as1, ad1, h1, z16, z128)

    h2, as2, ad2 = _dense2(op1[0], op1[1], dp1[0], dp1[1], EXP,
                           b1.reshape(1, HC1), W2, avs2, avd2)
    dp2, op2 = _edge2(src, dst, as2.reshape(N), ad2.reshape(N), h2, z1, z64)

    return _final(op2[0], op2[1], dp2[0].reshape(N, 1), dp2[1].reshape(N, 1),
                  b2.reshape(1, OUT), Wfc, bfc.reshape(1, 2))
